# baseline (speedup 1.0000x reference)
import functools

import jax
import jax.numpy as jnp
from jax import lax
from jax.experimental import pallas as pl
from jax.experimental.pallas import tpu as pltpu
from jax.experimental.pallas import tpu_sc as plsc

N = 10000
E = 320000
D = 128
H1 = 32
H2 = 16

NC = 2
NS = 16
NW = NC * NS
CHUNK = 125
CHUNKS = 80
EPW = CHUNKS * CHUNK
ROWS_PER_SUB = 632
N_PAD = NS * ROWS_PER_SUB


NBUF = 4
TROWS = 158


def _spmm_body(combine, table_hbm, src_hbm, dst_hbm, w_hbm, zeros_hbm,
               out_hbm, src_v, dst_v, w_v, g0, g1, g2, g3, s0, s1, s2, s3,
               tbuf0, tbuf1, table_sh, accum_sh,
               sg0, sg1, sg2, sg3, ss0, ss1, ss2, ss3):
    gbufs = [g0, g1, g2, g3]
    sbufs = [s0, s1, s2, s3]
    sems_g = [sg0, sg1, sg2, sg3]
    sems_s = [ss0, ss1, ss2, ss3]
    c = lax.axis_index("c")
    s = lax.axis_index("s")
    wid = s * NC + c
    base = s * ROWS_PER_SUB

    pltpu.sync_copy(src_hbm.at[wid], src_v)
    pltpu.sync_copy(dst_hbm.at[wid], dst_v)
    pltpu.sync_copy(w_hbm.at[wid], w_v)

    if combine:
        for t in range(ROWS_PER_SUB // TROWS):
            off = base + t * TROWS
            pltpu.sync_copy(table_hbm.at[0, pl.ds(off, TROWS)], tbuf0)
            pltpu.sync_copy(table_hbm.at[1, pl.ds(off, TROWS)], tbuf1)

            @plsc.parallel_loop(0, TROWS, 1, unroll=2)
            def _(r):
                t0 = tbuf0[r, pl.ds(0, 16)] + tbuf1[r, pl.ds(0, 16)]
                tbuf0[r, pl.ds(0, 16)] = jnp.maximum(t0, 0.0)
                t1 = tbuf0[r, pl.ds(16, 16)] + tbuf1[r, pl.ds(16, 16)]
                tbuf0[r, pl.ds(16, 16)] = jnp.maximum(t1, 0.0)

            pltpu.sync_copy(tbuf0, table_sh.at[pl.ds(off, TROWS)])
    else:
        pltpu.sync_copy(table_hbm.at[pl.ds(base, ROWS_PER_SUB)],
                        table_sh.at[pl.ds(base, ROWS_PER_SUB)])

    pltpu.sync_copy(zeros_hbm,
                    accum_sh.at[pl.ds(s * ROWS_PER_SUB, ROWS_PER_SUB)])
    plsc.subcore_barrier()

    def scale(j, gbuf, sbuf):
        jj = jnp.full((16,), j * CHUNK, jnp.int32)

        @plsc.parallel_loop(0, CHUNK, 1, unroll=5)
        def _(e):
            wv = plsc.load_gather(w_v, [jj + e])
            sbuf[e, pl.ds(0, 16)] = gbuf[e, pl.ds(0, 16)] * wv
            sbuf[e, pl.ds(16, 16)] = gbuf[e, pl.ds(16, 16)] * wv

    def gather_start(j, gbuf, sem):
        pltpu.async_copy(table_sh.at[src_v.at[j]], gbuf, sem)

    def gather_wait(j, gbuf, sem):
        pltpu.make_async_copy(table_sh.at[src_v.at[j]], gbuf, sem).wait()

    def scatter_start(j, sbuf, sem):
        pltpu.async_copy(sbuf, accum_sh.at[dst_v.at[j]], sem, add=True)

    def scatter_wait(j, sbuf, sem):
        pltpu.make_async_copy(sbuf, accum_sh.at[dst_v.at[j]], sem).wait()

    ROUNDS = CHUNKS // NBUF

    for b in range(NBUF):
        gather_start(b, gbufs[b], sems_g[b])

    def round_body(h, carry):
        for b in range(NBUF):
            j = h * NBUF + b
            gather_wait(j, gbufs[b], sems_g[b])

            @pl.when(h > 0)
            def _():
                scatter_wait(j - NBUF, sbufs[b], sems_s[b])

            scale(j, gbufs[b], sbufs[b])

            @pl.when(h < ROUNDS - 1)
            def _():
                gather_start(j + NBUF, gbufs[b], sems_g[b])

            scatter_start(j, sbufs[b], sems_s[b])
        return carry

    lax.fori_loop(0, ROUNDS, round_body, 0)
    for b in range(NBUF):
        scatter_wait(CHUNKS - NBUF + b, sbufs[b], sems_s[b])
    plsc.subcore_barrier()

    pltpu.sync_copy(accum_sh.at[pl.ds(s * ROWS_PER_SUB, ROWS_PER_SUB)],
                    out_hbm.at[c, pl.ds(s * ROWS_PER_SUB, ROWS_PER_SUB)])


def _spmm(table, srcp, dstp, wp, zeros, combine):
    mesh = plsc.VectorSubcoreMesh(core_axis_name="c", subcore_axis_name="s")
    f = pl.kernel(
        functools.partial(_spmm_body, combine),
        out_type=jax.ShapeDtypeStruct((NC, N_PAD, H1), jnp.float32),
        mesh=mesh,
        scratch_types=[
            pltpu.VMEM((CHUNKS, CHUNK), jnp.int32),
            pltpu.VMEM((CHUNKS, CHUNK), jnp.int32),
            pltpu.VMEM((EPW,), jnp.float32),
            pltpu.VMEM((CHUNK, H1), jnp.float32),
            pltpu.VMEM((CHUNK, H1), jnp.float32),
            pltpu.VMEM((CHUNK, H1), jnp.float32),
            pltpu.VMEM((CHUNK, H1), jnp.float32),
            pltpu.VMEM((CHUNK, H1), jnp.float32),
            pltpu.VMEM((CHUNK, H1), jnp.float32),
            pltpu.VMEM((CHUNK, H1), jnp.float32),
            pltpu.VMEM((CHUNK, H1), jnp.float32),
            pltpu.VMEM((TROWS, H1), jnp.float32),
            pltpu.VMEM((TROWS, H1), jnp.float32),
            pltpu.VMEM_SHARED((N_PAD, H1), jnp.float32),
            pltpu.VMEM_SHARED((N_PAD, H1), jnp.float32),
            pltpu.SemaphoreType.DMA,
            pltpu.SemaphoreType.DMA,
            pltpu.SemaphoreType.DMA,
            pltpu.SemaphoreType.DMA,
            pltpu.SemaphoreType.DMA,
            pltpu.SemaphoreType.DMA,
            pltpu.SemaphoreType.DMA,
            pltpu.SemaphoreType.DMA,
        ],
        compiler_params=pltpu.CompilerParams(
            needs_layout_passes=False, use_tc_tiling_on_sc=False),
    )
    return f(table, srcp, dstp, wp, zeros)


def _mm_body(x_ref, w_ref, o_ref):
    o_ref[pl.ds(0, N), :] = jax.lax.dot_general(
        x_ref[...], w_ref[...], (((1,), (0,)), ((), ())),
        preferred_element_type=jnp.float32)
    o_ref[pl.ds(N, N_PAD - N), :] = jnp.zeros((N_PAD - N, H1), jnp.float32)


def _z_body(q_ref, wm_ref, ws_ref, eps_ref, z_ref):
    g = q_ref[0, :N, :] + q_ref[1, :N, :]
    zm = jax.lax.dot_general(g, wm_ref[...], (((1,), (0,)), ((), ())),
                             preferred_element_type=jnp.float32)
    zl = jax.lax.dot_general(g, ws_ref[...], (((1,), (0,)), ((), ())),
                             preferred_element_type=jnp.float32)
    z_ref[...] = zm + eps_ref[...] * jnp.exp(zl)


def _dec_body(a_ref, b_ref, o_ref):
    m = jax.lax.dot_general(
        a_ref[...], b_ref[...], (((1,), (1,)), ((), ())),
        preferred_element_type=jnp.float32)
    for r in range(BM):
        o_ref[pl.ds(r * N, N)] = m[r, :]


BM = 256


@jax.jit
def kernel(x, edge_index, edge_weight, eps, W0, W_mean, W_std):
    srcp = edge_index[0].astype(jnp.int32).reshape(NW, CHUNKS, CHUNK)
    dstp = edge_index[1].astype(jnp.int32).reshape(NW, CHUNKS, CHUNK)
    wp = edge_weight.astype(jnp.float32).reshape(NW, EPW)
    zeros = jnp.zeros((ROWS_PER_SUB, H1), jnp.float32)

    h0 = pl.pallas_call(
        _mm_body,
        out_shape=jax.ShapeDtypeStruct((N_PAD, H1), jnp.float32),
    )(x, W0)

    p = _spmm(h0, srcp, dstp, wp, zeros, combine=False)
    q = _spmm(p, srcp, dstp, wp, zeros, combine=True)

    z = pl.pallas_call(
        _z_body,
        out_shape=jax.ShapeDtypeStruct((N, H2), jnp.float32),
    )(q, W_mean, W_std, eps)

    rec = pl.pallas_call(
        _dec_body,
        grid=(pl.cdiv(N, BM),),
        in_specs=[
            pl.BlockSpec((BM, H2), lambda i: (i, 0)),
            pl.BlockSpec((N, H2), lambda i: (0, 0)),
        ],
        out_specs=pl.BlockSpec((BM * N,), lambda i: (i,)),
        out_shape=jax.ShapeDtypeStruct((N * N,), jnp.float32),
    )(z, z)

    return rec


# speedup vs baseline: 1.0073x; 1.0073x over previous
import functools

import jax
import jax.numpy as jnp
from jax import lax
from jax.experimental import pallas as pl
from jax.experimental.pallas import tpu as pltpu
from jax.experimental.pallas import tpu_sc as plsc

N = 10000
E = 320000
D = 128
H1 = 32
H2 = 16

NC = 2
NS = 16
NW = NC * NS
CHUNK = 125
CHUNKS = 80
EPW = CHUNKS * CHUNK
ROWS_PER_SUB = 632
N_PAD = NS * ROWS_PER_SUB


NBUF = 4
TROWS = 158


def _spmm_body(combine, table_hbm, src_hbm, dst_hbm, w_hbm, zeros_hbm,
               out_hbm, src_v, dst_v, w_v, g0, g1, g2, g3, s0, s1, s2, s3,
               tbuf0, tbuf1, table_sh, accum_sh,
               sg0, sg1, sg2, sg3, ss0, ss1, ss2, ss3):
    gbufs = [g0, g1, g2, g3]
    sbufs = [s0, s1, s2, s3]
    sems_g = [sg0, sg1, sg2, sg3]
    sems_s = [ss0, ss1, ss2, ss3]
    c = lax.axis_index("c")
    s = lax.axis_index("s")
    wid = s * NC + c
    base = s * ROWS_PER_SUB

    pltpu.sync_copy(src_hbm.at[wid], src_v)
    pltpu.sync_copy(dst_hbm.at[wid], dst_v)
    pltpu.sync_copy(w_hbm.at[wid], w_v)

    if combine:
        for t in range(ROWS_PER_SUB // TROWS):
            off = base + t * TROWS
            pltpu.sync_copy(table_hbm.at[0, pl.ds(off, TROWS)], tbuf0)
            pltpu.sync_copy(table_hbm.at[1, pl.ds(off, TROWS)], tbuf1)

            @plsc.parallel_loop(0, TROWS, 1, unroll=2)
            def _(r):
                t0 = tbuf0[r, pl.ds(0, 16)] + tbuf1[r, pl.ds(0, 16)]
                tbuf0[r, pl.ds(0, 16)] = jnp.maximum(t0, 0.0)
                t1 = tbuf0[r, pl.ds(16, 16)] + tbuf1[r, pl.ds(16, 16)]
                tbuf0[r, pl.ds(16, 16)] = jnp.maximum(t1, 0.0)

            pltpu.sync_copy(tbuf0, table_sh.at[pl.ds(off, TROWS)])
    else:
        pltpu.sync_copy(table_hbm.at[pl.ds(base, ROWS_PER_SUB)],
                        table_sh.at[pl.ds(base, ROWS_PER_SUB)])

    pltpu.sync_copy(zeros_hbm,
                    accum_sh.at[pl.ds(s * ROWS_PER_SUB, ROWS_PER_SUB)])
    plsc.subcore_barrier()

    def scale(j, gbuf, sbuf):
        jj = jnp.full((16,), j * CHUNK, jnp.int32)

        @plsc.parallel_loop(0, CHUNK, 1, unroll=5)
        def _(e):
            wv = plsc.load_gather(w_v, [jj + e])
            sbuf[e, pl.ds(0, 16)] = gbuf[e, pl.ds(0, 16)] * wv
            sbuf[e, pl.ds(16, 16)] = gbuf[e, pl.ds(16, 16)] * wv

    def gather_start(j, gbuf, sem):
        pltpu.async_copy(table_sh.at[src_v.at[j]], gbuf, sem)

    def gather_wait(j, gbuf, sem):
        pltpu.make_async_copy(table_sh.at[src_v.at[j]], gbuf, sem).wait()

    def scatter_start(j, sbuf, sem):
        pltpu.async_copy(sbuf, accum_sh.at[dst_v.at[j]], sem, add=True)

    def scatter_wait(j, sbuf, sem):
        pltpu.make_async_copy(sbuf, accum_sh.at[dst_v.at[j]], sem).wait()

    ROUNDS = CHUNKS // NBUF

    for b in range(NBUF):
        gather_start(b, gbufs[b], sems_g[b])

    def round_body(h, carry):
        for b in range(NBUF):
            j = h * NBUF + b
            gather_wait(j, gbufs[b], sems_g[b])

            @pl.when(h > 0)
            def _():
                scatter_wait(j - NBUF, sbufs[b], sems_s[b])

            scale(j, gbufs[b], sbufs[b])

            @pl.when(h < ROUNDS - 1)
            def _():
                gather_start(j + NBUF, gbufs[b], sems_g[b])

            scatter_start(j, sbufs[b], sems_s[b])
        return carry

    lax.fori_loop(0, ROUNDS, round_body, 0)
    for b in range(NBUF):
        scatter_wait(CHUNKS - NBUF + b, sbufs[b], sems_s[b])
    plsc.subcore_barrier()

    pltpu.sync_copy(accum_sh.at[pl.ds(s * ROWS_PER_SUB, ROWS_PER_SUB)],
                    out_hbm.at[c, pl.ds(s * ROWS_PER_SUB, ROWS_PER_SUB)])


def _spmm(table, srcp, dstp, wp, zeros, combine):
    mesh = plsc.VectorSubcoreMesh(core_axis_name="c", subcore_axis_name="s")
    f = pl.kernel(
        functools.partial(_spmm_body, combine),
        out_type=jax.ShapeDtypeStruct((NC, N_PAD, H1), jnp.float32),
        mesh=mesh,
        scratch_types=[
            pltpu.VMEM((CHUNKS, CHUNK), jnp.int32),
            pltpu.VMEM((CHUNKS, CHUNK), jnp.int32),
            pltpu.VMEM((EPW,), jnp.float32),
            pltpu.VMEM((CHUNK, H1), jnp.float32),
            pltpu.VMEM((CHUNK, H1), jnp.float32),
            pltpu.VMEM((CHUNK, H1), jnp.float32),
            pltpu.VMEM((CHUNK, H1), jnp.float32),
            pltpu.VMEM((CHUNK, H1), jnp.float32),
            pltpu.VMEM((CHUNK, H1), jnp.float32),
            pltpu.VMEM((CHUNK, H1), jnp.float32),
            pltpu.VMEM((CHUNK, H1), jnp.float32),
            pltpu.VMEM((TROWS, H1), jnp.float32),
            pltpu.VMEM((TROWS, H1), jnp.float32),
            pltpu.VMEM_SHARED((N_PAD, H1), jnp.float32),
            pltpu.VMEM_SHARED((N_PAD, H1), jnp.float32),
            pltpu.SemaphoreType.DMA,
            pltpu.SemaphoreType.DMA,
            pltpu.SemaphoreType.DMA,
            pltpu.SemaphoreType.DMA,
            pltpu.SemaphoreType.DMA,
            pltpu.SemaphoreType.DMA,
            pltpu.SemaphoreType.DMA,
            pltpu.SemaphoreType.DMA,
        ],
        compiler_params=pltpu.CompilerParams(
            needs_layout_passes=False, use_tc_tiling_on_sc=False),
    )
    return f(table, srcp, dstp, wp, zeros)


def _mm_body(x_ref, w_ref, o_ref):
    o_ref[pl.ds(0, N), :] = jax.lax.dot_general(
        x_ref[...], w_ref[...], (((1,), (0,)), ((), ())),
        preferred_element_type=jnp.float32)
    o_ref[pl.ds(N, N_PAD - N), :] = jnp.zeros((N_PAD - N, H1), jnp.float32)


def _z_body(q_ref, wm_ref, ws_ref, eps_ref, z_ref):
    g = q_ref[0, :N, :] + q_ref[1, :N, :]
    zm = jax.lax.dot_general(g, wm_ref[...], (((1,), (0,)), ((), ())),
                             preferred_element_type=jnp.float32)
    zl = jax.lax.dot_general(g, ws_ref[...], (((1,), (0,)), ((), ())),
                             preferred_element_type=jnp.float32)
    z_ref[...] = zm + eps_ref[...] * jnp.exp(zl)


def _dec_body(a_ref, b_ref, o_ref):
    m = jax.lax.dot_general(
        a_ref[...], b_ref[...], (((1,), (1,)), ((), ())),
        preferred_element_type=jnp.float32)
    for r in range(BM):
        o_ref[pl.ds(r * N, N)] = m[r, :]


BM = 512


@jax.jit
def kernel(x, edge_index, edge_weight, eps, W0, W_mean, W_std):
    srcp = edge_index[0].astype(jnp.int32).reshape(NW, CHUNKS, CHUNK)
    dstp = edge_index[1].astype(jnp.int32).reshape(NW, CHUNKS, CHUNK)
    wp = edge_weight.astype(jnp.float32).reshape(NW, EPW)
    zeros = jnp.zeros((ROWS_PER_SUB, H1), jnp.float32)

    h0 = pl.pallas_call(
        _mm_body,
        out_shape=jax.ShapeDtypeStruct((N_PAD, H1), jnp.float32),
    )(x, W0)

    p = _spmm(h0, srcp, dstp, wp, zeros, combine=False)
    q = _spmm(p, srcp, dstp, wp, zeros, combine=True)

    z = pl.pallas_call(
        _z_body,
        out_shape=jax.ShapeDtypeStruct((N, H2), jnp.float32),
    )(q, W_mean, W_std, eps)

    rec = pl.pallas_call(
        _dec_body,
        grid=(pl.cdiv(N, BM),),
        in_specs=[
            pl.BlockSpec((BM, H2), lambda i: (i, 0)),
            pl.BlockSpec((N, H2), lambda i: (0, 0)),
        ],
        out_specs=pl.BlockSpec((BM * N,), lambda i: (i,)),
        out_shape=jax.ShapeDtypeStruct((N * N,), jnp.float32),
    )(z, z)

    return rec
